# baseline (speedup 1.0000x reference)
import functools

import jax
import jax.numpy as jnp
from jax import lax
from jax.experimental import pallas as pl
from jax.experimental.pallas import tpu as pltpu
from jax.experimental.pallas import tpu_sc as plsc

N_NODES = 10000
E = 160000
D = 128
D_VAL = 16
NUM_FES = 16
H1 = 512
H_FC = 64

NC = 2
NS = 16
NW = NC * NS

CHUNK = 128

ZROWS = 624
ZTAIL_OFF = ZROWS * NS
ZTAIL = N_NODES - ZTAIL_OFF

NSEG = 2
ESEG = E // NSEG


def _gather_body(n_chunks, hn_hbm, src_hbm, dst_hbm, hs_hbm, hd_hbm,
                 idx_a, rows_a, idx_b, rows_b, sem_a, sem_b):
    wid = lax.axis_index("s") * NC + lax.axis_index("c")
    iters = -(-n_chunks // NW)

    def step(t, _):
        chunk = t * NW + wid

        @pl.when(chunk < n_chunks)
        def _():
            base = chunk * CHUNK
            pltpu.sync_copy(src_hbm.at[pl.ds(base, CHUNK)], idx_a)
            pltpu.sync_copy(dst_hbm.at[pl.ds(base, CHUNK)], idx_b)
            cp_a = pltpu.async_copy(hn_hbm.at[idx_a], rows_a, sem_a)
            cp_b = pltpu.async_copy(hn_hbm.at[idx_b], rows_b, sem_b)
            cp_a.wait()
            pltpu.sync_copy(rows_a, hs_hbm.at[pl.ds(base, CHUNK)])
            cp_b.wait()
            pltpu.sync_copy(rows_b, hd_hbm.at[pl.ds(base, CHUNK)])
        return None

    lax.fori_loop(0, iters, step, None)


def _sc_gather(hn, src, dst):
    n = src.shape[0]
    mesh = plsc.VectorSubcoreMesh(core_axis_name="c", subcore_axis_name="s")
    return pl.kernel(
        functools.partial(_gather_body, n // CHUNK),
        out_type=(
            jax.ShapeDtypeStruct((n, D), jnp.float32),
            jax.ShapeDtypeStruct((n, D), jnp.float32),
        ),
        mesh=mesh,
        scratch_types=[
            pltpu.VMEM((CHUNK,), jnp.int32),
            pltpu.VMEM((CHUNK, D), jnp.float32),
            pltpu.VMEM((CHUNK,), jnp.int32),
            pltpu.VMEM((CHUNK, D), jnp.float32),
            pltpu.SemaphoreType.DMA,
            pltpu.SemaphoreType.DMA,
        ],
        name=f"sc_gather_{n}",
    )(hn, src, dst)


def _scatter_body(n_chunks, hen_s_hbm, dst_hbm, zero_hbm, out_hbm,
                  idx_v, rows_v, acc):
    cid = lax.axis_index("c")
    sid = lax.axis_index("s")
    wid = sid * NC + cid
    iters = -(-n_chunks // NW)

    r0 = sid * ZROWS
    pltpu.sync_copy(zero_hbm.at[pl.ds(r0, ZROWS)], acc.at[pl.ds(r0, ZROWS)])

    @pl.when(sid == 0)
    def _():
        pltpu.sync_copy(zero_hbm.at[pl.ds(ZTAIL_OFF, ZTAIL)],
                        acc.at[pl.ds(ZTAIL_OFF, ZTAIL)])
    plsc.subcore_barrier()

    def step(t, _):
        chunk = t * NW + wid

        @pl.when(chunk < n_chunks)
        def _():
            base = chunk * CHUNK
            pltpu.sync_copy(dst_hbm.at[pl.ds(base, CHUNK)], idx_v)
            pltpu.sync_copy(hen_s_hbm.at[pl.ds(base, CHUNK)], rows_v)
            pltpu.sync_copy(rows_v, acc.at[idx_v], add=True)
        return None

    lax.fori_loop(0, iters, step, None)
    plsc.subcore_barrier()
    pltpu.sync_copy(acc.at[pl.ds(r0, ZROWS)], out_hbm.at[cid, pl.ds(r0, ZROWS)])

    @pl.when(sid == 0)
    def _():
        pltpu.sync_copy(acc.at[pl.ds(ZTAIL_OFF, ZTAIL)],
                        out_hbm.at[cid, pl.ds(ZTAIL_OFF, ZTAIL)])


def _sc_scatter(hen_s, dst, zero):
    n = dst.shape[0]
    mesh = plsc.VectorSubcoreMesh(core_axis_name="c", subcore_axis_name="s")
    return pl.kernel(
        functools.partial(_scatter_body, n // CHUNK),
        out_type=jax.ShapeDtypeStruct((NC, N_NODES, D), jnp.float32),
        mesh=mesh,
        scratch_types=[
            pltpu.VMEM((CHUNK,), jnp.int32),
            pltpu.VMEM((CHUNK, D), jnp.float32),
            pltpu.VMEM_SHARED((N_NODES, D), jnp.float32),
        ],
        name=f"sc_scatter_{n}",
    )(hen_s, dst, zero)


BE = 800


def _edge_body(he, hs, hd, fe, fes, norm,
               w1a, w1b, w1c, b1, w2, b2, fcw1, fcw2,
               hen_out, hen_s_out):
    bf = jnp.bfloat16
    h1 = jnp.dot(he[...].astype(bf), w1a[...], preferred_element_type=jnp.float32)
    h1 += jnp.dot(hs[...].astype(bf), w1b[...], preferred_element_type=jnp.float32)
    h1 += jnp.dot(hd[...].astype(bf), w1c[...], preferred_element_type=jnp.float32)
    h1 = jnp.maximum(h1 + b1[...], 0.0)
    v = jnp.dot(h1.astype(bf), w2[...], preferred_element_type=jnp.float32) + b2[...]
    h = jnp.maximum(jnp.dot(fes[...].astype(bf), fcw1[...],
                            preferred_element_type=jnp.float32) * 0.25, 0.0)
    g = jnp.dot(h.astype(bf), fcw2[...], preferred_element_type=jnp.float32)
    acc = v[:, 0:1] * g[:, 0:D]
    for j in range(1, D_VAL):
        acc += v[:, j:j + 1] * g[:, j * D:(j + 1) * D]
    heu = fe[...] * acc * (1.0 / 32.0)
    hen = he[...] + heu
    hen_out[...] = hen
    hen_s_out[...] = hen * norm[...]


def _tc_edge(he, hs, hd, fe, fes, norm2d, w1a, w1b, w1c, b1, w2, b2,
             fcw1, fcw2):
    n = he.shape[0]
    grid = (n // BE,)
    eb = lambda w: pl.BlockSpec((BE, w), lambda i: (i, 0))
    full = lambda a, b: pl.BlockSpec((a, b), lambda i: (0, 0))
    return pl.pallas_call(
        _edge_body,
        grid=grid,
        in_specs=[
            eb(D), eb(D), eb(D), eb(1), eb(NUM_FES), eb(1),
            full(D, H1), full(D, H1), full(D, H1), full(1, H1),
            full(H1, D_VAL), full(1, D_VAL),
            full(NUM_FES, H_FC), full(H_FC, D_VAL * D),
        ],
        out_specs=[eb(D), eb(D)],
        out_shape=(
            jax.ShapeDtypeStruct((n, D), jnp.float32),
            jax.ShapeDtypeStruct((n, D), jnp.float32),
        ),
    )(he, hs, hd, fe, fes, norm2d, w1a, w1b, w1c, b1, w2, b2, fcw1, fcw2)


BN = 2000


def _node_body(hn, parts, w1a, w1b, b1, w2, b2, out):
    bf = jnp.bfloat16
    ntmp = parts[0]
    for s in range(1, NSEG * NC):
        ntmp += parts[s]
    h1 = jnp.dot(hn[...].astype(bf), w1a[...], preferred_element_type=jnp.float32)
    h1 += jnp.dot(ntmp.astype(bf), w1b[...], preferred_element_type=jnp.float32)
    h1 = jnp.maximum(h1 + b1[...], 0.0)
    out[...] = hn[...] + jnp.dot(h1.astype(bf), w2[...],
                                 preferred_element_type=jnp.float32) + b2[...]


def _tc_node(hn, parts, nu_W1, nu_b1, nu_W2, nu_b2):
    grid = (N_NODES // BN,)
    nb = pl.BlockSpec((BN, D), lambda i: (i, 0))
    full = lambda a, b: pl.BlockSpec((a, b), lambda i: (0, 0))
    return pl.pallas_call(
        _node_body,
        grid=grid,
        in_specs=[
            nb,
            pl.BlockSpec((NSEG * NC, BN, D), lambda i: (0, i, 0)),
            full(D, H1), full(D, H1), full(1, H1),
            full(H1, D), full(1, D),
        ],
        out_specs=nb,
        out_shape=jax.ShapeDtypeStruct((N_NODES, D), jnp.float32),
    )(hn, parts, nu_W1[:D].astype(jnp.bfloat16),
      nu_W1[D:].astype(jnp.bfloat16), nu_b1.reshape(1, H1),
      nu_W2.astype(jnp.bfloat16), nu_b2.reshape(1, D))


def kernel(hn, he, edge_index, fe, fes, norm, ev_W1, ev_b1, ev_W2, ev_b2,
           fc_W1, fc_W2, nu_W1, nu_b1, nu_W2, nu_b2):
    src = edge_index[0]
    dst = edge_index[1]
    norm2d = norm.reshape(E, 1)
    bf = jnp.bfloat16
    w1a = ev_W1[:D].astype(bf)
    w1b = ev_W1[D:2 * D].astype(bf)
    w1c = ev_W1[2 * D:].astype(bf)
    b1 = ev_b1.reshape(1, H1)
    w2 = ev_W2.astype(bf)
    b2 = ev_b2.reshape(1, D_VAL)
    fcw1 = fc_W1.astype(bf)
    fcw2 = fc_W2.astype(bf)
    zero = jnp.zeros((N_NODES, D), jnp.float32)

    hen_segs = []
    part_segs = []
    for s in range(NSEG):
        lo, hi = s * ESEG, (s + 1) * ESEG
        hs, hd = _sc_gather(hn, src[lo:hi], dst[lo:hi])
        hen, hen_s = _tc_edge(he[lo:hi], hs, hd, fe[lo:hi], fes[lo:hi],
                              norm2d[lo:hi], w1a, w1b, w1c, b1, w2, b2,
                              fcw1, fcw2)
        parts = _sc_scatter(hen_s, dst[lo:hi], zero)
        hen_segs.append(hen)
        part_segs.append(parts)

    hen_full = jnp.concatenate(hen_segs, axis=0)
    parts_all = jnp.concatenate(part_segs, axis=0)
    hnn = _tc_node(hn, parts_all, nu_W1, nu_b1, nu_W2, nu_b2)
    return (hnn, hen_full)


# speedup vs baseline: 1.6334x; 1.6334x over previous
import functools

import jax
import jax.numpy as jnp
from jax import lax
from jax.experimental import pallas as pl
from jax.experimental.pallas import tpu as pltpu
from jax.experimental.pallas import tpu_sc as plsc

N_NODES = 10000
E = 160000
D = 128
D_VAL = 16
NUM_FES = 16
H1 = 512
H_FC = 64

NC = 2
NS = 16
NW = NC * NS

CHUNK = 128

ZROWS = 624
ZTAIL_OFF = ZROWS * NS
ZTAIL = N_NODES - ZTAIL_OFF


def _gather_body(n_chunks, hn_hbm, src_hbm, dst_hbm, hs_hbm, hd_hbm,
                 idx_a, rows_a, idx_b, rows_b, sem_a, sem_b):
    wid = lax.axis_index("s") * NC + lax.axis_index("c")
    iters = -(-n_chunks // NW)

    def step(t, _):
        chunk = t * NW + wid

        @pl.when(chunk < n_chunks)
        def _():
            base = chunk * CHUNK
            pltpu.sync_copy(src_hbm.at[pl.ds(base, CHUNK)], idx_a)
            pltpu.sync_copy(dst_hbm.at[pl.ds(base, CHUNK)], idx_b)
            cp_a = pltpu.async_copy(hn_hbm.at[idx_a], rows_a, sem_a)
            cp_b = pltpu.async_copy(hn_hbm.at[idx_b], rows_b, sem_b)
            cp_a.wait()
            pltpu.sync_copy(rows_a, hs_hbm.at[pl.ds(base, CHUNK)])
            cp_b.wait()
            pltpu.sync_copy(rows_b, hd_hbm.at[pl.ds(base, CHUNK)])
        return None

    lax.fori_loop(0, iters, step, None)


def _sc_gather(hn, src, dst):
    n = src.shape[0]
    mesh = plsc.VectorSubcoreMesh(core_axis_name="c", subcore_axis_name="s")
    return pl.kernel(
        functools.partial(_gather_body, n // CHUNK),
        out_type=(
            jax.ShapeDtypeStruct((n, D), jnp.float32),
            jax.ShapeDtypeStruct((n, D), jnp.float32),
        ),
        mesh=mesh,
        scratch_types=[
            pltpu.VMEM((CHUNK,), jnp.int32),
            pltpu.VMEM((CHUNK, D), jnp.float32),
            pltpu.VMEM((CHUNK,), jnp.int32),
            pltpu.VMEM((CHUNK, D), jnp.float32),
            pltpu.SemaphoreType.DMA,
            pltpu.SemaphoreType.DMA,
        ],
        name=f"sc_gather_{n}",
    )(hn, src, dst)


def _scatter_body(n_chunks, hen_s_hbm, dst_hbm, zero_hbm, out_hbm,
                  idx_v, rows_v, acc):
    cid = lax.axis_index("c")
    sid = lax.axis_index("s")
    wid = sid * NC + cid
    iters = -(-n_chunks // NW)

    r0 = sid * ZROWS
    pltpu.sync_copy(zero_hbm.at[pl.ds(r0, ZROWS)], acc.at[pl.ds(r0, ZROWS)])

    @pl.when(sid == 0)
    def _():
        pltpu.sync_copy(zero_hbm.at[pl.ds(ZTAIL_OFF, ZTAIL)],
                        acc.at[pl.ds(ZTAIL_OFF, ZTAIL)])
    plsc.subcore_barrier()

    def step(t, _):
        chunk = t * NW + wid

        @pl.when(chunk < n_chunks)
        def _():
            base = chunk * CHUNK
            pltpu.sync_copy(dst_hbm.at[pl.ds(base, CHUNK)], idx_v)
            pltpu.sync_copy(hen_s_hbm.at[pl.ds(base, CHUNK)], rows_v)
            pltpu.sync_copy(rows_v, acc.at[idx_v], add=True)
        return None

    lax.fori_loop(0, iters, step, None)
    plsc.subcore_barrier()
    pltpu.sync_copy(acc.at[pl.ds(r0, ZROWS)], out_hbm.at[cid, pl.ds(r0, ZROWS)])

    @pl.when(sid == 0)
    def _():
        pltpu.sync_copy(acc.at[pl.ds(ZTAIL_OFF, ZTAIL)],
                        out_hbm.at[cid, pl.ds(ZTAIL_OFF, ZTAIL)])


def _sc_scatter(hen_s, dst, zero):
    n = dst.shape[0]
    mesh = plsc.VectorSubcoreMesh(core_axis_name="c", subcore_axis_name="s")
    return pl.kernel(
        functools.partial(_scatter_body, n // CHUNK),
        out_type=jax.ShapeDtypeStruct((NC, N_NODES, D), jnp.float32),
        mesh=mesh,
        scratch_types=[
            pltpu.VMEM((CHUNK,), jnp.int32),
            pltpu.VMEM((CHUNK, D), jnp.float32),
            pltpu.VMEM_SHARED((N_NODES, D), jnp.float32),
        ],
        name=f"sc_scatter_{n}",
    )(hen_s, dst, zero)


BE = 640
NB = BE // 128


def _edge_body(he, hs, hd, feT, fesT, normT,
               w1a, w1b, w1c, b1, w2, b2, fcw1T, w2pT,
               hen_out, hen_s_out):
    bf = jnp.bfloat16
    h1 = jnp.dot(he[...].astype(bf), w1a[...], preferred_element_type=jnp.float32)
    h1 += jnp.dot(hs[...].astype(bf), w1b[...], preferred_element_type=jnp.float32)
    h1 += jnp.dot(hd[...].astype(bf), w1c[...], preferred_element_type=jnp.float32)
    h1 = jnp.maximum(h1 + b1[...], 0.0)
    v = jnp.dot(h1.astype(bf), w2[...], preferred_element_type=jnp.float32) + b2[...]
    vT = v.T.astype(bf)
    hT = jnp.maximum(jnp.dot(fcw1T[...], fesT[...],
                             preferred_element_type=jnp.float32) * 0.25, 0.0)
    hTb = hT.astype(bf)
    ht_parts = [vT[j:j + 1, :] * hTb for j in range(D_VAL)]
    HT = jnp.concatenate(ht_parts, axis=0)
    accT = jnp.dot(w2pT[...], HT, preferred_element_type=jnp.float32)
    heuT = feT[...] * accT * (1.0 / 32.0)
    henT = he[...].T + heuT
    hen_out[...] = henT.T
    hen_s_out[...] = (henT * normT[...]).T


def _tc_edge(he, hs, hd, feT, fesT, normT, w1a, w1b, w1c, b1, w2, b2,
             fcw1T, w2pT):
    n = he.shape[0]
    grid = (n // BE,)
    eb = lambda w: pl.BlockSpec((BE, w), lambda i: (i, 0))
    tb = lambda a: pl.BlockSpec((a, BE), lambda i: (0, i))
    full = lambda a, b: pl.BlockSpec((a, b), lambda i: (0, 0))
    return pl.pallas_call(
        _edge_body,
        grid=grid,
        in_specs=[
            eb(D), eb(D), eb(D), tb(1), tb(NUM_FES), tb(1),
            full(D, H1), full(D, H1), full(D, H1), full(1, H1),
            full(H1, D_VAL), full(1, D_VAL),
            full(H_FC, NUM_FES), full(D, D_VAL * H_FC),
        ],
        out_specs=[eb(D), eb(D)],
        out_shape=(
            jax.ShapeDtypeStruct((n, D), jnp.float32),
            jax.ShapeDtypeStruct((n, D), jnp.float32),
        ),
    )(he, hs, hd, feT, fesT, normT, w1a, w1b, w1c, b1, w2, b2, fcw1T, w2pT)


BN = 2000


def _node_body(hn, p0, p1, w1a, w1b, b1, w2, b2, out):
    bf = jnp.bfloat16
    ntmp = p0[0] + p1[0]
    h1 = jnp.dot(hn[...].astype(bf), w1a[...], preferred_element_type=jnp.float32)
    h1 += jnp.dot(ntmp.astype(bf), w1b[...], preferred_element_type=jnp.float32)
    h1 = jnp.maximum(h1 + b1[...], 0.0)
    out[...] = hn[...] + jnp.dot(h1.astype(bf), w2[...],
                                 preferred_element_type=jnp.float32) + b2[...]


def _tc_node(hn, parts, nu_W1, nu_b1, nu_W2, nu_b2):
    grid = (N_NODES // BN,)
    nb = pl.BlockSpec((BN, D), lambda i: (i, 0))
    full = lambda a, b: pl.BlockSpec((a, b), lambda i: (0, 0))
    return pl.pallas_call(
        _node_body,
        grid=grid,
        in_specs=[
            nb,
            pl.BlockSpec((1, BN, D), lambda i: (0, i, 0)),
            pl.BlockSpec((1, BN, D), lambda i: (1, i, 0)),
            full(D, H1), full(D, H1), full(1, H1),
            full(H1, D), full(1, D),
        ],
        out_specs=nb,
        out_shape=jax.ShapeDtypeStruct((N_NODES, D), jnp.float32),
    )(hn, parts, parts, nu_W1[:D].astype(jnp.bfloat16),
      nu_W1[D:].astype(jnp.bfloat16), nu_b1.reshape(1, H1),
      nu_W2.astype(jnp.bfloat16), nu_b2.reshape(1, D))


def kernel(hn, he, edge_index, fe, fes, norm, ev_W1, ev_b1, ev_W2, ev_b2,
           fc_W1, fc_W2, nu_W1, nu_b1, nu_W2, nu_b2):
    src = edge_index[0]
    dst = edge_index[1]
    bf = jnp.bfloat16
    w1a = ev_W1[:D].astype(bf)
    w1b = ev_W1[D:2 * D].astype(bf)
    w1c = ev_W1[2 * D:].astype(bf)
    b1 = ev_b1.reshape(1, H1)
    w2 = ev_W2.astype(bf)
    b2 = ev_b2.reshape(1, D_VAL)
    fcw1T = fc_W1.T.astype(bf)
    w2pT = (fc_W2.reshape(H_FC, D_VAL, D)
            .transpose(2, 1, 0).reshape(D, D_VAL * H_FC).astype(bf))
    feT = fe.reshape(1, E)
    fesT = fes.T
    normT = norm.reshape(1, E)
    zero = jnp.zeros((N_NODES, D), jnp.float32)

    hs, hd = _sc_gather(hn, src, dst)
    hen, hen_s = _tc_edge(he, hs, hd, feT, fesT, normT,
                          w1a, w1b, w1c, b1, w2, b2, fcw1T, w2pT)
    parts = _sc_scatter(hen_s, dst, zero)
    hnn = _tc_node(hn, parts, nu_W1, nu_b1, nu_W2, nu_b2)
    return (hnn, hen)


# speedup vs baseline: 2.0594x; 1.2608x over previous
import functools

import jax
import jax.numpy as jnp
from jax import lax
from jax.experimental import pallas as pl
from jax.experimental.pallas import tpu as pltpu
from jax.experimental.pallas import tpu_sc as plsc

N_NODES = 10000
E = 160000
D = 128
D_VAL = 16
NUM_FES = 16
H1 = 512
H_FC = 64

NC = 2
NS = 16
NW = NC * NS

CHUNK = 128

ZROWS = 624
ZTAIL_OFF = ZROWS * NS
ZTAIL = N_NODES - ZTAIL_OFF


def _gather_body(n_chunks, hn_hbm, src_hbm, dst_hbm, hs_hbm, hd_hbm,
                 idx_a, rows_a, idx_b, rows_b, sem_a, sem_b):
    wid = lax.axis_index("s") * NC + lax.axis_index("c")
    iters = -(-n_chunks // NW)

    def step(t, _):
        chunk = t * NW + wid

        @pl.when(chunk < n_chunks)
        def _():
            base = chunk * CHUNK
            pltpu.sync_copy(src_hbm.at[pl.ds(base, CHUNK)], idx_a)
            pltpu.sync_copy(dst_hbm.at[pl.ds(base, CHUNK)], idx_b)
            cp_a = pltpu.async_copy(hn_hbm.at[idx_a], rows_a, sem_a)
            cp_b = pltpu.async_copy(hn_hbm.at[idx_b], rows_b, sem_b)
            cp_a.wait()
            pltpu.sync_copy(rows_a, hs_hbm.at[pl.ds(base, CHUNK)])
            cp_b.wait()
            pltpu.sync_copy(rows_b, hd_hbm.at[pl.ds(base, CHUNK)])
        return None

    lax.fori_loop(0, iters, step, None)


def _sc_gather(hn, src, dst):
    n = src.shape[0]
    mesh = plsc.VectorSubcoreMesh(core_axis_name="c", subcore_axis_name="s")
    return pl.kernel(
        functools.partial(_gather_body, n // CHUNK),
        out_type=(
            jax.ShapeDtypeStruct((n, D), jnp.float32),
            jax.ShapeDtypeStruct((n, D), jnp.float32),
        ),
        mesh=mesh,
        scratch_types=[
            pltpu.VMEM((CHUNK,), jnp.int32),
            pltpu.VMEM((CHUNK, D), jnp.float32),
            pltpu.VMEM((CHUNK,), jnp.int32),
            pltpu.VMEM((CHUNK, D), jnp.float32),
            pltpu.SemaphoreType.DMA,
            pltpu.SemaphoreType.DMA,
        ],
        name=f"sc_gather_{n}",
    )(hn, src, dst)


def _scatter_body(n_chunks, hen_s_hbm, dst_hbm, zero_hbm, out_hbm,
                  idx_v, rows_v, acc):
    cid = lax.axis_index("c")
    sid = lax.axis_index("s")
    wid = sid * NC + cid
    iters = -(-n_chunks // NW)

    r0 = sid * ZROWS
    pltpu.sync_copy(zero_hbm.at[pl.ds(r0, ZROWS)], acc.at[pl.ds(r0, ZROWS)])

    @pl.when(sid == 0)
    def _():
        pltpu.sync_copy(zero_hbm.at[pl.ds(ZTAIL_OFF, ZTAIL)],
                        acc.at[pl.ds(ZTAIL_OFF, ZTAIL)])
    plsc.subcore_barrier()

    def step(t, _):
        chunk = t * NW + wid

        @pl.when(chunk < n_chunks)
        def _():
            base = chunk * CHUNK
            pltpu.sync_copy(dst_hbm.at[pl.ds(base, CHUNK)], idx_v)
            pltpu.sync_copy(hen_s_hbm.at[pl.ds(base, CHUNK)], rows_v)
            pltpu.sync_copy(rows_v, acc.at[idx_v], add=True)
        return None

    lax.fori_loop(0, iters, step, None)
    plsc.subcore_barrier()
    pltpu.sync_copy(acc.at[pl.ds(r0, ZROWS)], out_hbm.at[cid, pl.ds(r0, ZROWS)])

    @pl.when(sid == 0)
    def _():
        pltpu.sync_copy(acc.at[pl.ds(ZTAIL_OFF, ZTAIL)],
                        out_hbm.at[cid, pl.ds(ZTAIL_OFF, ZTAIL)])


def _sc_scatter(hen_s, dst, zero):
    n = dst.shape[0]
    mesh = plsc.VectorSubcoreMesh(core_axis_name="c", subcore_axis_name="s")
    return pl.kernel(
        functools.partial(_scatter_body, n // CHUNK),
        out_type=jax.ShapeDtypeStruct((NC, N_NODES, D), jnp.float32),
        mesh=mesh,
        scratch_types=[
            pltpu.VMEM((CHUNK,), jnp.int32),
            pltpu.VMEM((CHUNK, D), jnp.float32),
            pltpu.VMEM_SHARED((N_NODES, D), jnp.float32),
        ],
        name=f"sc_scatter_{n}",
    )(hen_s, dst, zero)


BE = 6400
NB = BE // 128


def _edge_body(he, hs, hd, feT, fesT, normT,
               w1a, w1b, w1c, b1, w2, b2, fcw1T, w2pT,
               hen_out, hen_s_out):
    bf = jnp.bfloat16
    h1 = jnp.dot(he[...].astype(bf), w1a[...], preferred_element_type=jnp.float32)
    h1 += jnp.dot(hs[...].astype(bf), w1b[...], preferred_element_type=jnp.float32)
    h1 += jnp.dot(hd[...].astype(bf), w1c[...], preferred_element_type=jnp.float32)
    h1 = jnp.maximum(h1 + b1[...], 0.0)
    v = jnp.dot(h1.astype(bf), w2[...], preferred_element_type=jnp.float32) + b2[...]
    vT = v.T.astype(bf)
    hT = jnp.maximum(jnp.dot(fcw1T[...], fesT[...],
                             preferred_element_type=jnp.float32) * 0.25, 0.0)
    hTb = hT.astype(bf)
    ht_parts = [vT[j:j + 1, :] * hTb for j in range(D_VAL)]
    HT = jnp.concatenate(ht_parts, axis=0)
    accT = jnp.dot(w2pT[...], HT, preferred_element_type=jnp.float32)
    heuT = feT[...] * accT * (1.0 / 32.0)
    henT = he[...].T + heuT
    hen_out[...] = henT.T
    hen_s_out[...] = (henT * normT[...]).T


def _tc_edge(he, hs, hd, feT, fesT, normT, w1a, w1b, w1c, b1, w2, b2,
             fcw1T, w2pT):
    n = he.shape[0]
    grid = (n // BE,)
    eb = lambda w: pl.BlockSpec((BE, w), lambda i: (i, 0))
    tb = lambda a: pl.BlockSpec((a, BE), lambda i: (0, i))
    full = lambda a, b: pl.BlockSpec((a, b), lambda i: (0, 0))
    return pl.pallas_call(
        _edge_body,
        grid=grid,
        in_specs=[
            eb(D), eb(D), eb(D), tb(1), tb(NUM_FES), tb(1),
            full(D, H1), full(D, H1), full(D, H1), full(1, H1),
            full(H1, D_VAL), full(1, D_VAL),
            full(H_FC, NUM_FES), full(D, D_VAL * H_FC),
        ],
        out_specs=[eb(D), eb(D)],
        out_shape=(
            jax.ShapeDtypeStruct((n, D), jnp.float32),
            jax.ShapeDtypeStruct((n, D), jnp.float32),
        ),
    )(he, hs, hd, feT, fesT, normT, w1a, w1b, w1c, b1, w2, b2, fcw1T, w2pT)


BN = 2000


def _node_body(hn, p0, p1, w1a, w1b, b1, w2, b2, out):
    bf = jnp.bfloat16
    ntmp = p0[0] + p1[0]
    h1 = jnp.dot(hn[...].astype(bf), w1a[...], preferred_element_type=jnp.float32)
    h1 += jnp.dot(ntmp.astype(bf), w1b[...], preferred_element_type=jnp.float32)
    h1 = jnp.maximum(h1 + b1[...], 0.0)
    out[...] = hn[...] + jnp.dot(h1.astype(bf), w2[...],
                                 preferred_element_type=jnp.float32) + b2[...]


def _tc_node(hn, parts, nu_W1, nu_b1, nu_W2, nu_b2):
    grid = (N_NODES // BN,)
    nb = pl.BlockSpec((BN, D), lambda i: (i, 0))
    full = lambda a, b: pl.BlockSpec((a, b), lambda i: (0, 0))
    return pl.pallas_call(
        _node_body,
        grid=grid,
        in_specs=[
            nb,
            pl.BlockSpec((1, BN, D), lambda i: (0, i, 0)),
            pl.BlockSpec((1, BN, D), lambda i: (1, i, 0)),
            full(D, H1), full(D, H1), full(1, H1),
            full(H1, D), full(1, D),
        ],
        out_specs=nb,
        out_shape=jax.ShapeDtypeStruct((N_NODES, D), jnp.float32),
    )(hn, parts, parts, nu_W1[:D].astype(jnp.bfloat16),
      nu_W1[D:].astype(jnp.bfloat16), nu_b1.reshape(1, H1),
      nu_W2.astype(jnp.bfloat16), nu_b2.reshape(1, D))


def kernel(hn, he, edge_index, fe, fes, norm, ev_W1, ev_b1, ev_W2, ev_b2,
           fc_W1, fc_W2, nu_W1, nu_b1, nu_W2, nu_b2):
    src = edge_index[0]
    dst = edge_index[1]
    bf = jnp.bfloat16
    w1a = ev_W1[:D].astype(bf)
    w1b = ev_W1[D:2 * D].astype(bf)
    w1c = ev_W1[2 * D:].astype(bf)
    b1 = ev_b1.reshape(1, H1)
    w2 = ev_W2.astype(bf)
    b2 = ev_b2.reshape(1, D_VAL)
    fcw1T = fc_W1.T.astype(bf)
    w2pT = (fc_W2.reshape(H_FC, D_VAL, D)
            .transpose(2, 1, 0).reshape(D, D_VAL * H_FC).astype(bf))
    feT = fe.reshape(1, E)
    fesT = fes.T
    normT = norm.reshape(1, E)
    zero = jnp.zeros((N_NODES, D), jnp.float32)

    hs, hd = _sc_gather(hn, src, dst)
    hen, hen_s = _tc_edge(he, hs, hd, feT, fesT, normT,
                          w1a, w1b, w1c, b1, w2, b2, fcw1T, w2pT)
    parts = _sc_scatter(hen_s, dst, zero)
    hnn = _tc_node(hn, parts, nu_W1, nu_b1, nu_W2, nu_b2)
    return (hnn, hen)


# speedup vs baseline: 2.4708x; 1.1998x over previous
import functools

import jax
import jax.numpy as jnp
from jax import lax
from jax.experimental import pallas as pl
from jax.experimental.pallas import tpu as pltpu
from jax.experimental.pallas import tpu_sc as plsc

N_NODES = 10000
E = 160000
D = 128
D_VAL = 16
NUM_FES = 16
H1 = 512
H_FC = 64

NC = 2
NS = 16
NW = NC * NS

CHUNK = 128

ZROWS = 624
ZTAIL_OFF = ZROWS * NS
ZTAIL = N_NODES - ZTAIL_OFF


def _gather_body(n_chunks, hn_hbm, src_hbm, dst_hbm, hs_hbm, hd_hbm,
                 idx_a0, idx_a1, rows_a0, rows_a1,
                 idx_b0, idx_b1, rows_b0, rows_b1,
                 sa0, sa1, sb0, sb1):
    wid = lax.axis_index("s") * NC + lax.axis_index("c")
    iters = -(-n_chunks // NW)
    pairs = -(-iters // 2)
    idx_a = (idx_a0, idx_a1)
    idx_b = (idx_b0, idx_b1)
    rows_a = (rows_a0, rows_a1)
    rows_b = (rows_b0, rows_b1)
    sa = (sa0, sa1)
    sb = (sb0, sb1)

    def start(chunk, p):
        base = chunk * CHUNK
        pltpu.sync_copy(src_hbm.at[pl.ds(base, CHUNK)], idx_a[p])
        pltpu.sync_copy(dst_hbm.at[pl.ds(base, CHUNK)], idx_b[p])
        pltpu.async_copy(hn_hbm.at[idx_a[p]], rows_a[p], sa[p])
        pltpu.async_copy(hn_hbm.at[idx_b[p]], rows_b[p], sb[p])

    def drain(chunk, p):
        base = chunk * CHUNK
        pltpu.make_async_copy(hn_hbm.at[idx_a[p]], rows_a[p], sa[p]).wait()
        pltpu.sync_copy(rows_a[p], hs_hbm.at[pl.ds(base, CHUNK)])
        pltpu.make_async_copy(hn_hbm.at[idx_b[p]], rows_b[p], sb[p]).wait()
        pltpu.sync_copy(rows_b[p], hd_hbm.at[pl.ds(base, CHUNK)])

    c0 = wid

    @pl.when(c0 < n_chunks)
    def _():
        start(c0, 0)

    def step(tp, _):
        for ph in (0, 1):
            t = 2 * tp + ph
            cur = t * NW + wid
            nxt = cur + NW

            @pl.when((t + 1 < iters) & (nxt < n_chunks))
            def _():
                start(nxt, 1 - ph)

            @pl.when(cur < n_chunks)
            def _():
                drain(cur, ph)
        return None

    lax.fori_loop(0, pairs, step, None)


def _sc_gather(hn, src, dst):
    n = src.shape[0]
    mesh = plsc.VectorSubcoreMesh(core_axis_name="c", subcore_axis_name="s")
    return pl.kernel(
        functools.partial(_gather_body, n // CHUNK),
        out_type=(
            jax.ShapeDtypeStruct((n, D), jnp.float32),
            jax.ShapeDtypeStruct((n, D), jnp.float32),
        ),
        mesh=mesh,
        scratch_types=[
            pltpu.VMEM((CHUNK,), jnp.int32),
            pltpu.VMEM((CHUNK,), jnp.int32),
            pltpu.VMEM((CHUNK, D), jnp.float32),
            pltpu.VMEM((CHUNK, D), jnp.float32),
            pltpu.VMEM((CHUNK,), jnp.int32),
            pltpu.VMEM((CHUNK,), jnp.int32),
            pltpu.VMEM((CHUNK, D), jnp.float32),
            pltpu.VMEM((CHUNK, D), jnp.float32),
            pltpu.SemaphoreType.DMA,
            pltpu.SemaphoreType.DMA,
            pltpu.SemaphoreType.DMA,
            pltpu.SemaphoreType.DMA,
        ],
        name=f"sc_gather_{n}",
    )(hn, src, dst)


def _scatter_body(n_chunks, hen_s_hbm, dst_hbm, zero_hbm, out_hbm,
                  idx0, idx1, rows0, rows1, s0, s1, si0, si1, acc):
    cid = lax.axis_index("c")
    sid = lax.axis_index("s")
    wid = sid * NC + cid
    iters = -(-n_chunks // NW)
    pairs = -(-iters // 2)
    idx = (idx0, idx1)
    rows = (rows0, rows1)
    sems = (s0, s1)
    isems = (si0, si1)

    def start(chunk, p):
        base = chunk * CHUNK
        pltpu.async_copy(dst_hbm.at[pl.ds(base, CHUNK)], idx[p], isems[p])
        pltpu.async_copy(hen_s_hbm.at[pl.ds(base, CHUNK)], rows[p], sems[p])

    def drain(p):
        pltpu.make_async_copy(dst_hbm.at[pl.ds(0, CHUNK)], idx[p],
                              isems[p]).wait()
        pltpu.make_async_copy(hen_s_hbm.at[pl.ds(0, CHUNK)], rows[p],
                              sems[p]).wait()
        pltpu.sync_copy(rows[p], acc.at[idx[p]], add=True)

    c0 = wid

    @pl.when(c0 < n_chunks)
    def _():
        start(c0, 0)

    r0 = sid * ZROWS
    pltpu.sync_copy(zero_hbm.at[pl.ds(r0, ZROWS)], acc.at[pl.ds(r0, ZROWS)])

    @pl.when(sid == 0)
    def _():
        pltpu.sync_copy(zero_hbm.at[pl.ds(ZTAIL_OFF, ZTAIL)],
                        acc.at[pl.ds(ZTAIL_OFF, ZTAIL)])
    plsc.subcore_barrier()

    def step(tp, _):
        for ph in (0, 1):
            t = 2 * tp + ph
            cur = t * NW + wid
            nxt = cur + NW

            @pl.when((t + 1 < iters) & (nxt < n_chunks))
            def _():
                start(nxt, 1 - ph)

            @pl.when(cur < n_chunks)
            def _():
                drain(ph)
        return None

    lax.fori_loop(0, pairs, step, None)
    plsc.subcore_barrier()
    pltpu.sync_copy(acc.at[pl.ds(r0, ZROWS)], out_hbm.at[cid, pl.ds(r0, ZROWS)])

    @pl.when(sid == 0)
    def _():
        pltpu.sync_copy(acc.at[pl.ds(ZTAIL_OFF, ZTAIL)],
                        out_hbm.at[cid, pl.ds(ZTAIL_OFF, ZTAIL)])


def _sc_scatter(hen_s, dst, zero):
    n = dst.shape[0]
    mesh = plsc.VectorSubcoreMesh(core_axis_name="c", subcore_axis_name="s")
    return pl.kernel(
        functools.partial(_scatter_body, n // CHUNK),
        out_type=jax.ShapeDtypeStruct((NC, N_NODES, D), jnp.float32),
        mesh=mesh,
        scratch_types=[
            pltpu.VMEM((CHUNK,), jnp.int32),
            pltpu.VMEM((CHUNK,), jnp.int32),
            pltpu.VMEM((CHUNK, D), jnp.float32),
            pltpu.VMEM((CHUNK, D), jnp.float32),
            pltpu.SemaphoreType.DMA,
            pltpu.SemaphoreType.DMA,
            pltpu.SemaphoreType.DMA,
            pltpu.SemaphoreType.DMA,
            pltpu.VMEM_SHARED((N_NODES, D), jnp.float32),
        ],
        name=f"sc_scatter_{n}",
    )(hen_s, dst, zero)


BE = 6400
NB = BE // 128


def _edge_body(he, hs, hd, feT, fesT, normT,
               w1a, w1b, w1c, b1, w2, b2, fcw1T, w2pT,
               hen_out, hen_s_out):
    bf = jnp.bfloat16
    h1 = jnp.dot(he[...].astype(bf), w1a[...], preferred_element_type=jnp.float32)
    h1 += jnp.dot(hs[...].astype(bf), w1b[...], preferred_element_type=jnp.float32)
    h1 += jnp.dot(hd[...].astype(bf), w1c[...], preferred_element_type=jnp.float32)
    h1 = jnp.maximum(h1 + b1[...], 0.0)
    v = jnp.dot(h1.astype(bf), w2[...], preferred_element_type=jnp.float32) + b2[...]
    vT = v.T.astype(bf)
    hT = jnp.maximum(jnp.dot(fcw1T[...], fesT[...],
                             preferred_element_type=jnp.float32) * 0.25, 0.0)
    hTb = hT.astype(bf)
    ht_parts = [vT[j:j + 1, :] * hTb for j in range(D_VAL)]
    HT = jnp.concatenate(ht_parts, axis=0)
    accT = jnp.dot(w2pT[...], HT, preferred_element_type=jnp.float32)
    heuT = feT[...] * accT * (1.0 / 32.0)
    henT = he[...].T + heuT
    hen_out[...] = henT.T
    hen_s_out[...] = (henT * normT[...]).T


def _tc_edge(he, hs, hd, feT, fesT, normT, w1a, w1b, w1c, b1, w2, b2,
             fcw1T, w2pT):
    n = he.shape[0]
    grid = (n // BE,)
    eb = lambda w: pl.BlockSpec((BE, w), lambda i: (i, 0))
    tb = lambda a: pl.BlockSpec((a, BE), lambda i: (0, i))
    full = lambda a, b: pl.BlockSpec((a, b), lambda i: (0, 0))
    return pl.pallas_call(
        _edge_body,
        grid=grid,
        in_specs=[
            eb(D), eb(D), eb(D), tb(1), tb(NUM_FES), tb(1),
            full(D, H1), full(D, H1), full(D, H1), full(1, H1),
            full(H1, D_VAL), full(1, D_VAL),
            full(H_FC, NUM_FES), full(D, D_VAL * H_FC),
        ],
        out_specs=[eb(D), eb(D)],
        out_shape=(
            jax.ShapeDtypeStruct((n, D), jnp.float32),
            jax.ShapeDtypeStruct((n, D), jnp.float32),
        ),
    )(he, hs, hd, feT, fesT, normT, w1a, w1b, w1c, b1, w2, b2, fcw1T, w2pT)


BN = 2000


def _node_body(hn, p0, p1, w1a, w1b, b1, w2, b2, out):
    bf = jnp.bfloat16
    ntmp = p0[0] + p1[0]
    h1 = jnp.dot(hn[...].astype(bf), w1a[...], preferred_element_type=jnp.float32)
    h1 += jnp.dot(ntmp.astype(bf), w1b[...], preferred_element_type=jnp.float32)
    h1 = jnp.maximum(h1 + b1[...], 0.0)
    out[...] = hn[...] + jnp.dot(h1.astype(bf), w2[...],
                                 preferred_element_type=jnp.float32) + b2[...]


def _tc_node(hn, parts, nu_W1, nu_b1, nu_W2, nu_b2):
    grid = (N_NODES // BN,)
    nb = pl.BlockSpec((BN, D), lambda i: (i, 0))
    full = lambda a, b: pl.BlockSpec((a, b), lambda i: (0, 0))
    return pl.pallas_call(
        _node_body,
        grid=grid,
        in_specs=[
            nb,
            pl.BlockSpec((1, BN, D), lambda i: (0, i, 0)),
            pl.BlockSpec((1, BN, D), lambda i: (1, i, 0)),
            full(D, H1), full(D, H1), full(1, H1),
            full(H1, D), full(1, D),
        ],
        out_specs=nb,
        out_shape=jax.ShapeDtypeStruct((N_NODES, D), jnp.float32),
    )(hn, parts, parts, nu_W1[:D].astype(jnp.bfloat16),
      nu_W1[D:].astype(jnp.bfloat16), nu_b1.reshape(1, H1),
      nu_W2.astype(jnp.bfloat16), nu_b2.reshape(1, D))


def kernel(hn, he, edge_index, fe, fes, norm, ev_W1, ev_b1, ev_W2, ev_b2,
           fc_W1, fc_W2, nu_W1, nu_b1, nu_W2, nu_b2):
    src = edge_index[0]
    dst = edge_index[1]
    bf = jnp.bfloat16
    w1a = ev_W1[:D].astype(bf)
    w1b = ev_W1[D:2 * D].astype(bf)
    w1c = ev_W1[2 * D:].astype(bf)
    b1 = ev_b1.reshape(1, H1)
    w2 = ev_W2.astype(bf)
    b2 = ev_b2.reshape(1, D_VAL)
    fcw1T = fc_W1.T.astype(bf)
    w2pT = (fc_W2.reshape(H_FC, D_VAL, D)
            .transpose(2, 1, 0).reshape(D, D_VAL * H_FC).astype(bf))
    feT = fe.reshape(1, E)
    fesT = fes.T
    normT = norm.reshape(1, E)
    zero = jnp.zeros((N_NODES, D), jnp.float32)

    hs, hd = _sc_gather(hn, src, dst)
    hen, hen_s = _tc_edge(he, hs, hd, feT, fesT, normT,
                          w1a, w1b, w1c, b1, w2, b2, fcw1T, w2pT)
    parts = _sc_scatter(hen_s, dst, zero)
    hnn = _tc_node(hn, parts, nu_W1, nu_b1, nu_W2, nu_b2)
    return (hnn, hen)


# speedup vs baseline: 2.5516x; 1.0327x over previous
import functools

import jax
import jax.numpy as jnp
from jax import lax
from jax.experimental import pallas as pl
from jax.experimental.pallas import tpu as pltpu
from jax.experimental.pallas import tpu_sc as plsc

N_NODES = 10000
E = 160000
D = 128
D_VAL = 16
NUM_FES = 16
H1 = 512
H_FC = 64

NC = 2
NS = 16
NW = NC * NS

CHUNK = 128

ZROWS = 624
ZTAIL_OFF = ZROWS * NS
ZTAIL = N_NODES - ZTAIL_OFF


def _gather_body(n_chunks, seg_c0, hn_hbm, src_hbm, dst_hbm, hs_hbm, hd_hbm,
                 idx_a0, idx_a1, rows_a0, rows_a1,
                 idx_b0, idx_b1, rows_b0, rows_b1,
                 sa0, sa1, sb0, sb1):
    wid = lax.axis_index("s") * NC + lax.axis_index("c")
    iters = -(-n_chunks // NW)
    pairs = -(-iters // 2)
    idx_a = (idx_a0, idx_a1)
    idx_b = (idx_b0, idx_b1)
    rows_a = (rows_a0, rows_a1)
    rows_b = (rows_b0, rows_b1)
    sa = (sa0, sa1)
    sb = (sb0, sb1)

    def start(chunk, p):
        base = (seg_c0 + chunk) * CHUNK
        pltpu.sync_copy(src_hbm.at[pl.ds(base, CHUNK)], idx_a[p])
        pltpu.sync_copy(dst_hbm.at[pl.ds(base, CHUNK)], idx_b[p])
        pltpu.async_copy(hn_hbm.at[idx_a[p]], rows_a[p], sa[p])
        pltpu.async_copy(hn_hbm.at[idx_b[p]], rows_b[p], sb[p])

    def drain(chunk, p):
        base = chunk * CHUNK
        pltpu.make_async_copy(hn_hbm.at[idx_a[p]], rows_a[p], sa[p]).wait()
        pltpu.sync_copy(rows_a[p], hs_hbm.at[pl.ds(base, CHUNK)])
        pltpu.make_async_copy(hn_hbm.at[idx_b[p]], rows_b[p], sb[p]).wait()
        pltpu.sync_copy(rows_b[p], hd_hbm.at[pl.ds(base, CHUNK)])

    c0 = wid

    @pl.when(c0 < n_chunks)
    def _():
        start(c0, 0)

    def step(tp, _):
        for ph in (0, 1):
            t = 2 * tp + ph
            cur = t * NW + wid
            nxt = cur + NW

            @pl.when((t + 1 < iters) & (nxt < n_chunks))
            def _():
                start(nxt, 1 - ph)

            @pl.when(cur < n_chunks)
            def _():
                drain(cur, ph)
        return None

    lax.fori_loop(0, pairs, step, None)


def _sc_gather(hn, src, dst, lo, n):
    mesh = plsc.VectorSubcoreMesh(core_axis_name="c", subcore_axis_name="s")
    return pl.kernel(
        functools.partial(_gather_body, n // CHUNK, lo // CHUNK),
        out_type=(
            jax.ShapeDtypeStruct((n, D), jnp.float32),
            jax.ShapeDtypeStruct((n, D), jnp.float32),
        ),
        mesh=mesh,
        scratch_types=[
            pltpu.VMEM((CHUNK,), jnp.int32),
            pltpu.VMEM((CHUNK,), jnp.int32),
            pltpu.VMEM((CHUNK, D), jnp.float32),
            pltpu.VMEM((CHUNK, D), jnp.float32),
            pltpu.VMEM((CHUNK,), jnp.int32),
            pltpu.VMEM((CHUNK,), jnp.int32),
            pltpu.VMEM((CHUNK, D), jnp.float32),
            pltpu.VMEM((CHUNK, D), jnp.float32),
            pltpu.SemaphoreType.DMA,
            pltpu.SemaphoreType.DMA,
            pltpu.SemaphoreType.DMA,
            pltpu.SemaphoreType.DMA,
        ],
        name=f"sc_gather_{lo}_{n}",
    )(hn, src, dst)


def _scatter_body(n_chunks, seg_c0, hen_s_hbm, dst_hbm, zero_hbm, out_hbm,
                  idx0, idx1, rows0, rows1, s0, s1, si0, si1, acc):
    cid = lax.axis_index("c")
    sid = lax.axis_index("s")
    wid = sid * NC + cid
    iters = -(-n_chunks // NW)
    pairs = -(-iters // 2)
    idx = (idx0, idx1)
    rows = (rows0, rows1)
    sems = (s0, s1)
    isems = (si0, si1)

    def start(chunk, p):
        base = chunk * CHUNK
        pltpu.async_copy(dst_hbm.at[pl.ds((seg_c0 + chunk) * CHUNK, CHUNK)],
                         idx[p], isems[p])
        pltpu.async_copy(hen_s_hbm.at[pl.ds(base, CHUNK)], rows[p], sems[p])

    def drain(p):
        pltpu.make_async_copy(dst_hbm.at[pl.ds(0, CHUNK)], idx[p],
                              isems[p]).wait()
        pltpu.make_async_copy(hen_s_hbm.at[pl.ds(0, CHUNK)], rows[p],
                              sems[p]).wait()
        pltpu.sync_copy(rows[p], acc.at[idx[p]], add=True)

    c0 = wid

    @pl.when(c0 < n_chunks)
    def _():
        start(c0, 0)

    r0 = sid * ZROWS
    pltpu.sync_copy(zero_hbm.at[pl.ds(r0, ZROWS)], acc.at[pl.ds(r0, ZROWS)])

    @pl.when(sid == 0)
    def _():
        pltpu.sync_copy(zero_hbm.at[pl.ds(ZTAIL_OFF, ZTAIL)],
                        acc.at[pl.ds(ZTAIL_OFF, ZTAIL)])
    plsc.subcore_barrier()

    def step(tp, _):
        for ph in (0, 1):
            t = 2 * tp + ph
            cur = t * NW + wid
            nxt = cur + NW

            @pl.when((t + 1 < iters) & (nxt < n_chunks))
            def _():
                start(nxt, 1 - ph)

            @pl.when(cur < n_chunks)
            def _():
                drain(ph)
        return None

    lax.fori_loop(0, pairs, step, None)
    plsc.subcore_barrier()
    pltpu.sync_copy(acc.at[pl.ds(r0, ZROWS)], out_hbm.at[cid, pl.ds(r0, ZROWS)])

    @pl.when(sid == 0)
    def _():
        pltpu.sync_copy(acc.at[pl.ds(ZTAIL_OFF, ZTAIL)],
                        out_hbm.at[cid, pl.ds(ZTAIL_OFF, ZTAIL)])


def _sc_scatter(hen_s, dst, zero, lo, n):
    mesh = plsc.VectorSubcoreMesh(core_axis_name="c", subcore_axis_name="s")
    return pl.kernel(
        functools.partial(_scatter_body, n // CHUNK, lo // CHUNK),
        out_type=jax.ShapeDtypeStruct((NC, N_NODES, D), jnp.float32),
        mesh=mesh,
        scratch_types=[
            pltpu.VMEM((CHUNK,), jnp.int32),
            pltpu.VMEM((CHUNK,), jnp.int32),
            pltpu.VMEM((CHUNK, D), jnp.float32),
            pltpu.VMEM((CHUNK, D), jnp.float32),
            pltpu.SemaphoreType.DMA,
            pltpu.SemaphoreType.DMA,
            pltpu.SemaphoreType.DMA,
            pltpu.SemaphoreType.DMA,
            pltpu.VMEM_SHARED((N_NODES, D), jnp.float32),
        ],
        name=f"sc_scatter_{lo}_{n}",
    )(hen_s, dst, zero)


BE = 3200
NB = BE // 128


def _edge_body(he, hs, hd, feT, fesT, normT,
               w1a, w1b, w1c, b1, w2, b2, fcw1T, w2pT,
               hen_out, hen_s_out):
    bf = jnp.bfloat16
    h1 = jnp.dot(he[...].astype(bf), w1a[...], preferred_element_type=jnp.float32)
    h1 += jnp.dot(hs[...].astype(bf), w1b[...], preferred_element_type=jnp.float32)
    h1 += jnp.dot(hd[...].astype(bf), w1c[...], preferred_element_type=jnp.float32)
    h1 = jnp.maximum(h1 + b1[...], 0.0)
    v = jnp.dot(h1.astype(bf), w2[...], preferred_element_type=jnp.float32) + b2[...]
    vT = v.T.astype(bf)
    hT = jnp.maximum(jnp.dot(fcw1T[...], fesT[...],
                             preferred_element_type=jnp.float32) * 0.25, 0.0)
    hTb = hT.astype(bf)
    ht_parts = [vT[j:j + 1, :] * hTb for j in range(D_VAL)]
    HT = jnp.concatenate(ht_parts, axis=0)
    accT = jnp.dot(w2pT[...], HT, preferred_element_type=jnp.float32)
    heuT = feT[...] * accT * (1.0 / 32.0)
    henT = he[...].T + heuT
    hen_out[...] = henT.T
    hen_s_out[...] = (henT * normT[...]).T


def _tc_edge(he, hs, hd, feT, fesT, normT, w1a, w1b, w1c, b1, w2, b2,
             fcw1T, w2pT, lo, n):
    sb = lo // BE
    grid = (n // BE,)
    eb = lambda w: pl.BlockSpec((BE, w), lambda i: (i, 0))
    ebo = lambda w: pl.BlockSpec((BE, w), lambda i: (i + sb, 0))
    tbo = lambda a: pl.BlockSpec((a, BE), lambda i: (0, i + sb))
    full = lambda a, b: pl.BlockSpec((a, b), lambda i: (0, 0))
    return pl.pallas_call(
        _edge_body,
        grid=grid,
        in_specs=[
            ebo(D), eb(D), eb(D), tbo(1), tbo(NUM_FES), tbo(1),
            full(D, H1), full(D, H1), full(D, H1), full(1, H1),
            full(H1, D_VAL), full(1, D_VAL),
            full(H_FC, NUM_FES), full(D, D_VAL * H_FC),
        ],
        out_specs=[eb(D), eb(D)],
        out_shape=(
            jax.ShapeDtypeStruct((n, D), jnp.float32),
            jax.ShapeDtypeStruct((n, D), jnp.float32),
        ),
    )(he, hs, hd, feT, fesT, normT, w1a, w1b, w1c, b1, w2, b2, fcw1T, w2pT)


BN = 2000


def _node_body(hn, p0, p1, p2, p3, w1a, w1b, b1, w2, b2, out):
    bf = jnp.bfloat16
    ntmp = p0[0] + p1[0] + p2[0] + p3[0]
    h1 = jnp.dot(hn[...].astype(bf), w1a[...], preferred_element_type=jnp.float32)
    h1 += jnp.dot(ntmp.astype(bf), w1b[...], preferred_element_type=jnp.float32)
    h1 = jnp.maximum(h1 + b1[...], 0.0)
    out[...] = hn[...] + jnp.dot(h1.astype(bf), w2[...],
                                 preferred_element_type=jnp.float32) + b2[...]


def _tc_node(hn, parts0, parts1, nu_W1, nu_b1, nu_W2, nu_b2):
    grid = (N_NODES // BN,)
    nb = pl.BlockSpec((BN, D), lambda i: (i, 0))
    full = lambda a, b: pl.BlockSpec((a, b), lambda i: (0, 0))
    return pl.pallas_call(
        _node_body,
        grid=grid,
        in_specs=[
            nb,
            pl.BlockSpec((1, BN, D), lambda i: (0, i, 0)),
            pl.BlockSpec((1, BN, D), lambda i: (1, i, 0)),
            pl.BlockSpec((1, BN, D), lambda i: (0, i, 0)),
            pl.BlockSpec((1, BN, D), lambda i: (1, i, 0)),
            full(D, H1), full(D, H1), full(1, H1),
            full(H1, D), full(1, D),
        ],
        out_specs=nb,
        out_shape=jax.ShapeDtypeStruct((N_NODES, D), jnp.float32),
    )(hn, parts0, parts0, parts1, parts1, nu_W1[:D].astype(jnp.bfloat16),
      nu_W1[D:].astype(jnp.bfloat16), nu_b1.reshape(1, H1),
      nu_W2.astype(jnp.bfloat16), nu_b2.reshape(1, D))


def kernel(hn, he, edge_index, fe, fes, norm, ev_W1, ev_b1, ev_W2, ev_b2,
           fc_W1, fc_W2, nu_W1, nu_b1, nu_W2, nu_b2):
    src = edge_index[0]
    dst = edge_index[1]
    bf = jnp.bfloat16
    w1a = ev_W1[:D].astype(bf)
    w1b = ev_W1[D:2 * D].astype(bf)
    w1c = ev_W1[2 * D:].astype(bf)
    b1 = ev_b1.reshape(1, H1)
    w2 = ev_W2.astype(bf)
    b2 = ev_b2.reshape(1, D_VAL)
    fcw1T = fc_W1.T.astype(bf)
    w2pT = (fc_W2.reshape(H_FC, D_VAL, D)
            .transpose(2, 1, 0).reshape(D, D_VAL * H_FC).astype(bf))
    feT = fe.reshape(1, E)
    fesT = fes.T
    normT = norm.reshape(1, E)
    zero = jnp.zeros((N_NODES, D), jnp.float32)

    ESEG = E // 2
    hen_segs = []
    part_segs = []
    for lo in (0, ESEG):
        hs, hd = _sc_gather(hn, src, dst, lo, ESEG)
        hen, hen_s = _tc_edge(he, hs, hd, feT, fesT, normT,
                              w1a, w1b, w1c, b1, w2, b2, fcw1T, w2pT,
                              lo, ESEG)
        part_segs.append(_sc_scatter(hen_s, dst, zero, lo, ESEG))
        hen_segs.append(hen)

    hen_full = jnp.concatenate(hen_segs, axis=0)
    hnn = _tc_node(hn, part_segs[0], part_segs[1],
                   nu_W1, nu_b1, nu_W2, nu_b2)
    return (hnn, hen_full)


# speedup vs baseline: 2.5610x; 1.0037x over previous
import functools

import jax
import jax.numpy as jnp
from jax import lax
from jax.experimental import pallas as pl
from jax.experimental.pallas import tpu as pltpu
from jax.experimental.pallas import tpu_sc as plsc

N_NODES = 10000
E = 160000
D = 128
D_VAL = 16
NUM_FES = 16
H1 = 512
H_FC = 64

NC = 2
NS = 16
NW = NC * NS

CHUNK = 128

ZROWS = 624
ZTAIL_OFF = ZROWS * NS
ZTAIL = N_NODES - ZTAIL_OFF


def _gather_body(n_chunks, seg_c0, hn_hbm, src_hbm, dst_hbm, hs_hbm, hd_hbm,
                 idx_a0, idx_a1, rows_a0, rows_a1,
                 idx_b0, idx_b1, rows_b0, rows_b1,
                 sa0, sa1, sb0, sb1):
    wid = lax.axis_index("s") * NC + lax.axis_index("c")
    iters = -(-n_chunks // NW)
    pairs = -(-iters // 2)
    idx_a = (idx_a0, idx_a1)
    idx_b = (idx_b0, idx_b1)
    rows_a = (rows_a0, rows_a1)
    rows_b = (rows_b0, rows_b1)
    sa = (sa0, sa1)
    sb = (sb0, sb1)

    def start(chunk, p):
        base = (seg_c0 + chunk) * CHUNK
        pltpu.sync_copy(src_hbm.at[pl.ds(base, CHUNK)], idx_a[p])
        pltpu.sync_copy(dst_hbm.at[pl.ds(base, CHUNK)], idx_b[p])
        pltpu.async_copy(hn_hbm.at[idx_a[p]], rows_a[p], sa[p])
        pltpu.async_copy(hn_hbm.at[idx_b[p]], rows_b[p], sb[p])

    def drain(chunk, p):
        base = chunk * CHUNK
        pltpu.make_async_copy(hn_hbm.at[idx_a[p]], rows_a[p], sa[p]).wait()
        pltpu.sync_copy(rows_a[p], hs_hbm.at[pl.ds(base, CHUNK)])
        pltpu.make_async_copy(hn_hbm.at[idx_b[p]], rows_b[p], sb[p]).wait()
        pltpu.sync_copy(rows_b[p], hd_hbm.at[pl.ds(base, CHUNK)])

    c0 = wid

    @pl.when(c0 < n_chunks)
    def _():
        start(c0, 0)

    def step(tp, _):
        for ph in (0, 1):
            t = 2 * tp + ph
            cur = t * NW + wid
            nxt = cur + NW

            @pl.when((t + 1 < iters) & (nxt < n_chunks))
            def _():
                start(nxt, 1 - ph)

            @pl.when(cur < n_chunks)
            def _():
                drain(cur, ph)
        return None

    lax.fori_loop(0, pairs, step, None)


def _sc_gather(hn, src, dst, lo, n):
    mesh = plsc.VectorSubcoreMesh(core_axis_name="c", subcore_axis_name="s")
    return pl.kernel(
        functools.partial(_gather_body, n // CHUNK, lo // CHUNK),
        out_type=(
            jax.ShapeDtypeStruct((n, D), jnp.float32),
            jax.ShapeDtypeStruct((n, D), jnp.float32),
        ),
        mesh=mesh,
        scratch_types=[
            pltpu.VMEM((CHUNK,), jnp.int32),
            pltpu.VMEM((CHUNK,), jnp.int32),
            pltpu.VMEM((CHUNK, D), jnp.float32),
            pltpu.VMEM((CHUNK, D), jnp.float32),
            pltpu.VMEM((CHUNK,), jnp.int32),
            pltpu.VMEM((CHUNK,), jnp.int32),
            pltpu.VMEM((CHUNK, D), jnp.float32),
            pltpu.VMEM((CHUNK, D), jnp.float32),
            pltpu.SemaphoreType.DMA,
            pltpu.SemaphoreType.DMA,
            pltpu.SemaphoreType.DMA,
            pltpu.SemaphoreType.DMA,
        ],
        name=f"sc_gather_{lo}_{n}",
    )(hn, src, dst)


def _scatter_body(n_chunks, seg_c0, hen_s_hbm, dst_hbm, zero_hbm, out_hbm,
                  idx0, idx1, rows0, rows1, s0, s1, si0, si1, acc):
    cid = lax.axis_index("c")
    sid = lax.axis_index("s")
    wid = sid * NC + cid
    iters = -(-n_chunks // NW)
    pairs = -(-iters // 2)
    idx = (idx0, idx1)
    rows = (rows0, rows1)
    sems = (s0, s1)
    isems = (si0, si1)

    def start(chunk, p):
        base = chunk * CHUNK
        pltpu.async_copy(dst_hbm.at[pl.ds((seg_c0 + chunk) * CHUNK, CHUNK)],
                         idx[p], isems[p])
        pltpu.async_copy(hen_s_hbm.at[pl.ds(base, CHUNK)], rows[p], sems[p])

    def drain(p):
        pltpu.make_async_copy(dst_hbm.at[pl.ds(0, CHUNK)], idx[p],
                              isems[p]).wait()
        pltpu.make_async_copy(hen_s_hbm.at[pl.ds(0, CHUNK)], rows[p],
                              sems[p]).wait()
        pltpu.sync_copy(rows[p], acc.at[idx[p]], add=True)

    c0 = wid

    @pl.when(c0 < n_chunks)
    def _():
        start(c0, 0)

    r0 = sid * ZROWS
    pltpu.sync_copy(zero_hbm.at[pl.ds(r0, ZROWS)], acc.at[pl.ds(r0, ZROWS)])

    @pl.when(sid == 0)
    def _():
        pltpu.sync_copy(zero_hbm.at[pl.ds(ZTAIL_OFF, ZTAIL)],
                        acc.at[pl.ds(ZTAIL_OFF, ZTAIL)])
    plsc.subcore_barrier()

    def step(tp, _):
        for ph in (0, 1):
            t = 2 * tp + ph
            cur = t * NW + wid
            nxt = cur + NW

            @pl.when((t + 1 < iters) & (nxt < n_chunks))
            def _():
                start(nxt, 1 - ph)

            @pl.when(cur < n_chunks)
            def _():
                drain(ph)
        return None

    lax.fori_loop(0, pairs, step, None)
    plsc.subcore_barrier()
    pltpu.sync_copy(acc.at[pl.ds(r0, ZROWS)], out_hbm.at[cid, pl.ds(r0, ZROWS)])

    @pl.when(sid == 0)
    def _():
        pltpu.sync_copy(acc.at[pl.ds(ZTAIL_OFF, ZTAIL)],
                        out_hbm.at[cid, pl.ds(ZTAIL_OFF, ZTAIL)])


def _sc_scatter(hen_s, dst, zero, lo, n):
    mesh = plsc.VectorSubcoreMesh(core_axis_name="c", subcore_axis_name="s")
    return pl.kernel(
        functools.partial(_scatter_body, n // CHUNK, lo // CHUNK),
        out_type=jax.ShapeDtypeStruct((NC, N_NODES, D), jnp.float32),
        mesh=mesh,
        scratch_types=[
            pltpu.VMEM((CHUNK,), jnp.int32),
            pltpu.VMEM((CHUNK,), jnp.int32),
            pltpu.VMEM((CHUNK, D), jnp.float32),
            pltpu.VMEM((CHUNK, D), jnp.float32),
            pltpu.SemaphoreType.DMA,
            pltpu.SemaphoreType.DMA,
            pltpu.SemaphoreType.DMA,
            pltpu.SemaphoreType.DMA,
            pltpu.VMEM_SHARED((N_NODES, D), jnp.float32),
        ],
        name=f"sc_scatter_{lo}_{n}",
    )(hen_s, dst, zero)


BE = 6400
NB = BE // 128


def _edge_body(he, hs, hd, feT, fesT, normT,
               w1a, w1b, w1c, b1, w2, b2, fcw1T, w2pT,
               hen_out, hen_s_out):
    bf = jnp.bfloat16
    h1 = jnp.dot(he[...].astype(bf), w1a[...], preferred_element_type=jnp.float32)
    h1 += jnp.dot(hs[...].astype(bf), w1b[...], preferred_element_type=jnp.float32)
    h1 += jnp.dot(hd[...].astype(bf), w1c[...], preferred_element_type=jnp.float32)
    h1 = jnp.maximum(h1 + b1[...], 0.0)
    v = jnp.dot(h1.astype(bf), w2[...], preferred_element_type=jnp.float32) + b2[...]
    vT = v.T.astype(bf)
    hT = jnp.maximum(jnp.dot(fcw1T[...], fesT[...],
                             preferred_element_type=jnp.float32) * 0.25, 0.0)
    hTb = hT.astype(bf)
    ht_parts = [vT[j:j + 1, :] * hTb for j in range(D_VAL)]
    HT = jnp.concatenate(ht_parts, axis=0)
    accT = jnp.dot(w2pT[...], HT, preferred_element_type=jnp.float32)
    heuT = feT[...] * accT * (1.0 / 32.0)
    henT = he[...].T + heuT
    hen_out[...] = henT.T
    hen_s_out[...] = (henT * normT[...]).T


def _tc_edge(he, hs, hd, feT, fesT, normT, w1a, w1b, w1c, b1, w2, b2,
             fcw1T, w2pT, lo, n):
    sb = lo // BE
    grid = (n // BE,)
    eb = lambda w: pl.BlockSpec((BE, w), lambda i: (i, 0))
    ebo = lambda w: pl.BlockSpec((BE, w), lambda i: (i + sb, 0))
    tbo = lambda a: pl.BlockSpec((a, BE), lambda i: (0, i + sb))
    full = lambda a, b: pl.BlockSpec((a, b), lambda i: (0, 0))
    return pl.pallas_call(
        _edge_body,
        grid=grid,
        in_specs=[
            ebo(D), eb(D), eb(D), tbo(1), tbo(NUM_FES), tbo(1),
            full(D, H1), full(D, H1), full(D, H1), full(1, H1),
            full(H1, D_VAL), full(1, D_VAL),
            full(H_FC, NUM_FES), full(D, D_VAL * H_FC),
        ],
        out_specs=[eb(D), eb(D)],
        out_shape=(
            jax.ShapeDtypeStruct((n, D), jnp.float32),
            jax.ShapeDtypeStruct((n, D), jnp.float32),
        ),
    )(he, hs, hd, feT, fesT, normT, w1a, w1b, w1c, b1, w2, b2, fcw1T, w2pT)


BN = 2000


def _node_body(hn, p0, p1, p2, p3, w1a, w1b, b1, w2, b2, out):
    bf = jnp.bfloat16
    ntmp = p0[0] + p1[0] + p2[0] + p3[0]
    h1 = jnp.dot(hn[...].astype(bf), w1a[...], preferred_element_type=jnp.float32)
    h1 += jnp.dot(ntmp.astype(bf), w1b[...], preferred_element_type=jnp.float32)
    h1 = jnp.maximum(h1 + b1[...], 0.0)
    out[...] = hn[...] + jnp.dot(h1.astype(bf), w2[...],
                                 preferred_element_type=jnp.float32) + b2[...]


def _tc_node(hn, parts0, parts1, nu_W1, nu_b1, nu_W2, nu_b2):
    grid = (N_NODES // BN,)
    nb = pl.BlockSpec((BN, D), lambda i: (i, 0))
    full = lambda a, b: pl.BlockSpec((a, b), lambda i: (0, 0))
    return pl.pallas_call(
        _node_body,
        grid=grid,
        in_specs=[
            nb,
            pl.BlockSpec((1, BN, D), lambda i: (0, i, 0)),
            pl.BlockSpec((1, BN, D), lambda i: (1, i, 0)),
            pl.BlockSpec((1, BN, D), lambda i: (0, i, 0)),
            pl.BlockSpec((1, BN, D), lambda i: (1, i, 0)),
            full(D, H1), full(D, H1), full(1, H1),
            full(H1, D), full(1, D),
        ],
        out_specs=nb,
        out_shape=jax.ShapeDtypeStruct((N_NODES, D), jnp.float32),
    )(hn, parts0, parts0, parts1, parts1, nu_W1[:D].astype(jnp.bfloat16),
      nu_W1[D:].astype(jnp.bfloat16), nu_b1.reshape(1, H1),
      nu_W2.astype(jnp.bfloat16), nu_b2.reshape(1, D))


def kernel(hn, he, edge_index, fe, fes, norm, ev_W1, ev_b1, ev_W2, ev_b2,
           fc_W1, fc_W2, nu_W1, nu_b1, nu_W2, nu_b2):
    src = edge_index[0]
    dst = edge_index[1]
    bf = jnp.bfloat16
    w1a = ev_W1[:D].astype(bf)
    w1b = ev_W1[D:2 * D].astype(bf)
    w1c = ev_W1[2 * D:].astype(bf)
    b1 = ev_b1.reshape(1, H1)
    w2 = ev_W2.astype(bf)
    b2 = ev_b2.reshape(1, D_VAL)
    fcw1T = fc_W1.T.astype(bf)
    w2pT = (fc_W2.reshape(H_FC, D_VAL, D)
            .transpose(2, 1, 0).reshape(D, D_VAL * H_FC).astype(bf))
    feT = fe.reshape(1, E)
    fesT = fes.T
    normT = norm.reshape(1, E)
    zero = jnp.zeros((N_NODES, D), jnp.float32)

    hen_segs = []
    part_segs = []
    for lo, n in ((0, 64000), (64000, 96000)):
        hs, hd = _sc_gather(hn, src, dst, lo, n)
        hen, hen_s = _tc_edge(he, hs, hd, feT, fesT, normT,
                              w1a, w1b, w1c, b1, w2, b2, fcw1T, w2pT,
                              lo, n)
        part_segs.append(_sc_scatter(hen_s, dst, zero, lo, n))
        hen_segs.append(hen)

    hen_full = jnp.concatenate(hen_segs, axis=0)
    hnn = _tc_node(hn, part_segs[0], part_segs[1],
                   nu_W1, nu_b1, nu_W2, nu_b2)
    return (hnn, hen_full)


# speedup vs baseline: 2.6388x; 1.0304x over previous
import functools

import jax
import jax.numpy as jnp
from jax import lax
from jax.experimental import pallas as pl
from jax.experimental.pallas import tpu as pltpu
from jax.experimental.pallas import tpu_sc as plsc

N_NODES = 10000
E = 160000
D = 128
D_VAL = 16
NUM_FES = 16
H1 = 512
H_FC = 64

NC = 2
NS = 16
NW = NC * NS

CHUNK = 128

ZROWS = 624
ZTAIL_OFF = ZROWS * NS
ZTAIL = N_NODES - ZTAIL_OFF


def _gather_body(n_chunks, seg_c0, hn_hbm, src_hbm, dst_hbm, hs_hbm, hd_hbm,
                 idx_a0, idx_a1, rows_a0, rows_a1,
                 idx_b0, idx_b1, rows_b0, rows_b1,
                 sa0, sa1, sb0, sb1):
    wid = lax.axis_index("s") * NC + lax.axis_index("c")
    iters = -(-n_chunks // NW)
    pairs = -(-iters // 2)
    idx_a = (idx_a0, idx_a1)
    idx_b = (idx_b0, idx_b1)
    rows_a = (rows_a0, rows_a1)
    rows_b = (rows_b0, rows_b1)
    sa = (sa0, sa1)
    sb = (sb0, sb1)

    def start(chunk, p):
        base = (seg_c0 + chunk) * CHUNK
        pltpu.sync_copy(src_hbm.at[pl.ds(base, CHUNK)], idx_a[p])
        pltpu.sync_copy(dst_hbm.at[pl.ds(base, CHUNK)], idx_b[p])
        pltpu.async_copy(hn_hbm.at[idx_a[p]], rows_a[p], sa[p])
        pltpu.async_copy(hn_hbm.at[idx_b[p]], rows_b[p], sb[p])

    def drain(chunk, p):
        base = chunk * CHUNK
        pltpu.make_async_copy(hn_hbm.at[idx_a[p]], rows_a[p], sa[p]).wait()
        pltpu.sync_copy(rows_a[p], hs_hbm.at[pl.ds(base, CHUNK)])
        pltpu.make_async_copy(hn_hbm.at[idx_b[p]], rows_b[p], sb[p]).wait()
        pltpu.sync_copy(rows_b[p], hd_hbm.at[pl.ds(base, CHUNK)])

    c0 = wid

    @pl.when(c0 < n_chunks)
    def _():
        start(c0, 0)

    def step(tp, _):
        for ph in (0, 1):
            t = 2 * tp + ph
            cur = t * NW + wid
            nxt = cur + NW

            @pl.when((t + 1 < iters) & (nxt < n_chunks))
            def _():
                start(nxt, 1 - ph)

            @pl.when(cur < n_chunks)
            def _():
                drain(cur, ph)
        return None

    lax.fori_loop(0, pairs, step, None)


def _sc_gather(hn, src, dst, lo, n):
    mesh = plsc.VectorSubcoreMesh(core_axis_name="c", subcore_axis_name="s")
    return pl.kernel(
        functools.partial(_gather_body, n // CHUNK, lo // CHUNK),
        out_type=(
            jax.ShapeDtypeStruct((n, D), jnp.float32),
            jax.ShapeDtypeStruct((n, D), jnp.float32),
        ),
        mesh=mesh,
        scratch_types=[
            pltpu.VMEM((CHUNK,), jnp.int32),
            pltpu.VMEM((CHUNK,), jnp.int32),
            pltpu.VMEM((CHUNK, D), jnp.float32),
            pltpu.VMEM((CHUNK, D), jnp.float32),
            pltpu.VMEM((CHUNK,), jnp.int32),
            pltpu.VMEM((CHUNK,), jnp.int32),
            pltpu.VMEM((CHUNK, D), jnp.float32),
            pltpu.VMEM((CHUNK, D), jnp.float32),
            pltpu.SemaphoreType.DMA,
            pltpu.SemaphoreType.DMA,
            pltpu.SemaphoreType.DMA,
            pltpu.SemaphoreType.DMA,
        ],
        name=f"sc_gather_{lo}_{n}",
    )(hn, src, dst)


def _scatter_body(n_chunks, seg_c0, hen_s_hbm, dst_hbm, zero_hbm, out_hbm,
                  idx0, idx1, rows0, rows1, s0, s1, si0, si1, acc):
    cid = lax.axis_index("c")
    sid = lax.axis_index("s")
    wid = sid * NC + cid
    iters = -(-n_chunks // NW)
    pairs = -(-iters // 2)
    idx = (idx0, idx1)
    rows = (rows0, rows1)
    sems = (s0, s1)
    isems = (si0, si1)

    def start(chunk, p):
        base = chunk * CHUNK
        pltpu.async_copy(dst_hbm.at[pl.ds((seg_c0 + chunk) * CHUNK, CHUNK)],
                         idx[p], isems[p])
        pltpu.async_copy(hen_s_hbm.at[pl.ds(base, CHUNK)], rows[p], sems[p])

    def drain(p):
        pltpu.make_async_copy(dst_hbm.at[pl.ds(0, CHUNK)], idx[p],
                              isems[p]).wait()
        pltpu.make_async_copy(hen_s_hbm.at[pl.ds(0, CHUNK)], rows[p],
                              sems[p]).wait()
        pltpu.sync_copy(rows[p], acc.at[idx[p]], add=True)

    c0 = wid

    @pl.when(c0 < n_chunks)
    def _():
        start(c0, 0)

    r0 = sid * ZROWS
    pltpu.sync_copy(zero_hbm.at[pl.ds(r0, ZROWS)], acc.at[pl.ds(r0, ZROWS)])

    @pl.when(sid == 0)
    def _():
        pltpu.sync_copy(zero_hbm.at[pl.ds(ZTAIL_OFF, ZTAIL)],
                        acc.at[pl.ds(ZTAIL_OFF, ZTAIL)])
    plsc.subcore_barrier()

    def step(tp, _):
        for ph in (0, 1):
            t = 2 * tp + ph
            cur = t * NW + wid
            nxt = cur + NW

            @pl.when((t + 1 < iters) & (nxt < n_chunks))
            def _():
                start(nxt, 1 - ph)

            @pl.when(cur < n_chunks)
            def _():
                drain(ph)
        return None

    lax.fori_loop(0, pairs, step, None)
    plsc.subcore_barrier()
    pltpu.sync_copy(acc.at[pl.ds(r0, ZROWS)], out_hbm.at[cid, pl.ds(r0, ZROWS)])

    @pl.when(sid == 0)
    def _():
        pltpu.sync_copy(acc.at[pl.ds(ZTAIL_OFF, ZTAIL)],
                        out_hbm.at[cid, pl.ds(ZTAIL_OFF, ZTAIL)])


def _sc_scatter(hen_s, dst, zero, lo, n):
    mesh = plsc.VectorSubcoreMesh(core_axis_name="c", subcore_axis_name="s")
    return pl.kernel(
        functools.partial(_scatter_body, n // CHUNK, lo // CHUNK),
        out_type=jax.ShapeDtypeStruct((NC, N_NODES, D), jnp.float32),
        mesh=mesh,
        scratch_types=[
            pltpu.VMEM((CHUNK,), jnp.int32),
            pltpu.VMEM((CHUNK,), jnp.int32),
            pltpu.VMEM((CHUNK, D), jnp.float32),
            pltpu.VMEM((CHUNK, D), jnp.float32),
            pltpu.SemaphoreType.DMA,
            pltpu.SemaphoreType.DMA,
            pltpu.SemaphoreType.DMA,
            pltpu.SemaphoreType.DMA,
            pltpu.VMEM_SHARED((N_NODES, D), jnp.float32),
        ],
        name=f"sc_scatter_{lo}_{n}",
    )(hen_s, dst, zero)


BE = 6400
NB = BE // 128


def _edge_body(he, hs, hd, feT, fesT, normT,
               w1a, w1b, w1c, b1, w2, b2, fcw1T, w2pT,
               hen_out, hen_s_out):
    bf = jnp.bfloat16
    h1 = jnp.dot(he[...].astype(bf), w1a[...], preferred_element_type=jnp.float32)
    h1 += jnp.dot(hs[...].astype(bf), w1b[...], preferred_element_type=jnp.float32)
    h1 += jnp.dot(hd[...].astype(bf), w1c[...], preferred_element_type=jnp.float32)
    h1 = jnp.maximum(h1 + b1[...], 0.0)
    v = jnp.dot(h1.astype(bf), w2[...], preferred_element_type=jnp.float32) + b2[...]
    vT = v.T.astype(bf)
    hT = jnp.maximum(jnp.dot(fcw1T[...], fesT[...],
                             preferred_element_type=jnp.float32) * 0.25, 0.0)
    hTb = hT.astype(bf)
    ht_parts = [vT[j:j + 1, :] * hTb for j in range(D_VAL)]
    HT = jnp.concatenate(ht_parts, axis=0)
    accT = jnp.dot(w2pT[...], HT, preferred_element_type=jnp.float32)
    heuT = feT[...] * accT * (1.0 / 32.0)
    henT = he[...].T + heuT
    hen_out[...] = henT.T
    hen_s_out[...] = (henT * normT[...]).T


def _tc_edge(he, hs, hd, feT, fesT, normT, w1a, w1b, w1c, b1, w2, b2,
             fcw1T, w2pT, lo, n):
    sb = lo // BE
    grid = (n // BE,)
    eb = lambda w: pl.BlockSpec((BE, w), lambda i: (i, 0))
    ebo = lambda w: pl.BlockSpec((BE, w), lambda i: (i + sb, 0))
    tbo = lambda a: pl.BlockSpec((a, BE), lambda i: (0, i + sb))
    full = lambda a, b: pl.BlockSpec((a, b), lambda i: (0, 0))
    return pl.pallas_call(
        _edge_body,
        grid=grid,
        in_specs=[
            ebo(D), eb(D), eb(D), tbo(1), tbo(NUM_FES), tbo(1),
            full(D, H1), full(D, H1), full(D, H1), full(1, H1),
            full(H1, D_VAL), full(1, D_VAL),
            full(H_FC, NUM_FES), full(D, D_VAL * H_FC),
        ],
        out_specs=[eb(D), eb(D)],
        out_shape=(
            jax.ShapeDtypeStruct((n, D), jnp.float32),
            jax.ShapeDtypeStruct((n, D), jnp.float32),
        ),
    )(he, hs, hd, feT, fesT, normT, w1a, w1b, w1c, b1, w2, b2, fcw1T, w2pT)


BN = 2000


def _node_body(hn, p0, p1, p2, p3, w1a, w1b, b1, w2, b2, out):
    bf = jnp.bfloat16
    ntmp = p0[0] + p1[0] + p2[0] + p3[0]
    h1 = jnp.dot(hn[...].astype(bf), w1a[...], preferred_element_type=jnp.float32)
    h1 += jnp.dot(ntmp.astype(bf), w1b[...], preferred_element_type=jnp.float32)
    h1 = jnp.maximum(h1 + b1[...], 0.0)
    out[...] = hn[...] + jnp.dot(h1.astype(bf), w2[...],
                                 preferred_element_type=jnp.float32) + b2[...]


def _tc_node(hn, parts0, parts1, nu_W1, nu_b1, nu_W2, nu_b2):
    grid = (N_NODES // BN,)
    nb = pl.BlockSpec((BN, D), lambda i: (i, 0))
    full = lambda a, b: pl.BlockSpec((a, b), lambda i: (0, 0))
    return pl.pallas_call(
        _node_body,
        grid=grid,
        in_specs=[
            nb,
            pl.BlockSpec((1, BN, D), lambda i: (0, i, 0)),
            pl.BlockSpec((1, BN, D), lambda i: (1, i, 0)),
            pl.BlockSpec((1, BN, D), lambda i: (0, i, 0)),
            pl.BlockSpec((1, BN, D), lambda i: (1, i, 0)),
            full(D, H1), full(D, H1), full(1, H1),
            full(H1, D), full(1, D),
        ],
        out_specs=nb,
        out_shape=jax.ShapeDtypeStruct((N_NODES, D), jnp.float32),
    )(hn, parts0, parts0, parts1, parts1, nu_W1[:D].astype(jnp.bfloat16),
      nu_W1[D:].astype(jnp.bfloat16), nu_b1.reshape(1, H1),
      nu_W2.astype(jnp.bfloat16), nu_b2.reshape(1, D))


def kernel(hn, he, edge_index, fe, fes, norm, ev_W1, ev_b1, ev_W2, ev_b2,
           fc_W1, fc_W2, nu_W1, nu_b1, nu_W2, nu_b2):
    src = edge_index[0]
    dst = edge_index[1]
    bf = jnp.bfloat16
    w1a = ev_W1[:D].astype(bf)
    w1b = ev_W1[D:2 * D].astype(bf)
    w1c = ev_W1[2 * D:].astype(bf)
    b1 = ev_b1.reshape(1, H1)
    w2 = ev_W2.astype(bf)
    b2 = ev_b2.reshape(1, D_VAL)
    fcw1T = fc_W1.T.astype(bf)
    w2pT = (fc_W2.reshape(H_FC, D_VAL, D)
            .transpose(2, 1, 0).reshape(D, D_VAL * H_FC).astype(bf))
    feT = fe.reshape(1, E)
    fesT = fes.T
    normT = norm.reshape(1, E)
    zero = jnp.zeros((N_NODES, D), jnp.float32)

    hen_segs = []
    part_segs = []
    for lo, n in ((0, 64000), (64000, 96000)):
        hs, hd = _sc_gather(hn, src, dst, lo, n)
        if hen_segs:
            w2pT_s = lax.optimization_barrier((w2pT, hen_segs[-1]))[0]
        else:
            w2pT_s = w2pT
        hen, hen_s = _tc_edge(he, hs, hd, feT, fesT, normT,
                              w1a, w1b, w1c, b1, w2, b2, fcw1T, w2pT_s,
                              lo, n)
        part_segs.append(_sc_scatter(hen_s, dst, zero, lo, n))
        hen_segs.append(hen)

    hen_full = jnp.concatenate(hen_segs, axis=0)
    hnn = _tc_node(hn, part_segs[0], part_segs[1],
                   nu_W1, nu_b1, nu_W2, nu_b2)
    return (hnn, hen_full)
